# baseline (speedup 1.0000x reference)
import jax
import jax.numpy as jnp
from jax.experimental import pallas as pl
from jax.experimental.pallas import tpu as pltpu

M, K, N = 1024, 100000, 16
BM = 32
NBUF = 3
NBLK = M // BM


def _matmul_kernel(x_hbm, wt_ref, o_ref, bufs, sems):
    def copy_in(b, s):
        return pltpu.make_async_copy(
            x_hbm.at[pl.ds(b * BM, BM), :], bufs.at[s], sems.at[s])

    for b in range(NBUF):
        copy_in(b, b).start()

    def body(b, _):
        s = jax.lax.rem(b, NBUF)
        copy_in(b, s).wait()

        o_ref[pl.ds(b * BM, BM), :] = jax.lax.dot_general(
            bufs[s], wt_ref[...],
            dimension_numbers=(((1,), (1,)), ((), ())),
            preferred_element_type=jnp.float32)

        @pl.when(b + NBUF < NBLK)
        def _prefetch():
            copy_in(b + NBUF, s).start()

        return 0

    jax.lax.fori_loop(0, NBLK, body, 0)


def kernel(x, W):
    wt = W.T
    return pl.pallas_call(
        _matmul_kernel,
        in_specs=[
            pl.BlockSpec(memory_space=pltpu.MemorySpace.HBM),
            pl.BlockSpec((N, K), lambda: (0, 0)),
        ],
        out_specs=pl.BlockSpec((M, N), lambda: (0, 0)),
        out_shape=jax.ShapeDtypeStruct((M, N), jnp.float32),
        scratch_shapes=[
            pltpu.VMEM((NBUF, BM, K), jnp.float32),
            pltpu.SemaphoreType.DMA((NBUF,)),
        ],
    )(x, wt)
